# baseline (speedup 1.0000x reference)
import jax
import jax.numpy as jnp
from jax import lax
from jax.experimental import pallas as pl
from jax.experimental.pallas import tpu as pltpu
from jax.experimental.pallas import tpu_sc as plsc

N_NODES = 10000
N_EDGES = 320000
D = 128

NC = 2
NS = 16
EPW = N_EDGES // (NC * NS)
CHUNK = 50
NCHUNK = EPW // CHUNK
NBUF = 4
PD = 2
WSTRIDE = 128
NMAIN = (NCHUNK // NBUF) * NBUF
ROWS_PER_SUB = 624
ROWS_TAIL = N_NODES - NS * ROWS_PER_SUB


def _sc_aggregate(x, src4, dst4, w4, zeros):
    mesh = plsc.VectorSubcoreMesh(core_axis_name="c", subcore_axis_name="s")

    def body(x_hbm, src_hbm, dst_hbm, w_hbm, z_hbm, part_hbm,
             src_v, dst_v, wflat, rows, accum,
             g0, g1, g2, g3, s0, s1, s2, s3):
        G = [g0, g1, g2, g3]
        S = [s0, s1, s2, s3]
        cid = lax.axis_index("c")
        sid = lax.axis_index("s")

        pltpu.sync_copy(z_hbm.at[pl.ds(sid * ROWS_PER_SUB, ROWS_PER_SUB)],
                        accum.at[pl.ds(sid * ROWS_PER_SUB, ROWS_PER_SUB)])

        @pl.when(sid == NS - 1)
        def _():
            pltpu.sync_copy(z_hbm.at[pl.ds(NS * ROWS_PER_SUB, ROWS_TAIL)],
                            accum.at[pl.ds(NS * ROWS_PER_SUB, ROWS_TAIL)])

        plsc.subcore_barrier()

        pltpu.sync_copy(src_hbm.at[cid, sid], src_v)
        pltpu.sync_copy(dst_hbm.at[cid, sid], dst_v)

        def gather_start(c, b):
            pltpu.async_copy(x_hbm.at[src_v.at[c]], rows.at[b], G[b])
            pltpu.async_copy(w_hbm.at[cid, sid, c],
                             wflat.at[pl.ds(b * WSTRIDE, CHUNK)], G[b])

        def gather_wait(b):
            pltpu.make_async_copy(
                x_hbm.at[pl.ds(0, CHUNK)], rows.at[b], G[b]).wait()
            pltpu.make_async_copy(
                w_hbm.at[0, 0, 0], wflat.at[pl.ds(b * WSTRIDE, CHUNK)],
                G[b]).wait()

        def scatter_start(c, b):
            pltpu.async_copy(rows.at[b], accum.at[dst_v.at[c]], S[b], add=True)

        def scatter_wait(b):
            pltpu.make_async_copy(
                rows.at[b], accum.at[pl.ds(0, CHUNK)], S[b]).wait()

        def step(cc, b, prefetch):
            gather_wait(b)
            if prefetch:
                b2 = (b + PD) % NBUF

                @pl.when(cc + PD < NCHUNK)
                def _():
                    @pl.when(cc + PD >= NBUF)
                    def _():
                        scatter_wait(b2)
                    gather_start(cc + PD, b2)

            rb = rows.at[b]

            @pl.loop(0, CHUNK, unroll=5)
            def edge_loop(e):
                ie = jnp.full((16,), b * WSTRIDE + e, jnp.int32)
                wvec = plsc.load_gather(wflat, [ie])
                for j in range(D // 16):
                    seg = rb[e, pl.ds(j * 16, 16)]
                    rb[e, pl.ds(j * 16, 16)] = seg * wvec

            scatter_start(cc, b)

        for p in range(PD):
            gather_start(p, p)

        @pl.loop(0, NMAIN, step=NBUF)
        def chunk_loop(c):
            for b in range(NBUF):
                step(c + b, b, True)

        for cc in range(NMAIN, NCHUNK):
            step(cc, cc % NBUF, True)

        for b in range(NBUF):
            scatter_wait(b)

        plsc.subcore_barrier()
        pltpu.sync_copy(accum.at[pl.ds(sid * ROWS_PER_SUB, ROWS_PER_SUB)],
                        part_hbm.at[cid, pl.ds(sid * ROWS_PER_SUB, ROWS_PER_SUB)])

        @pl.when(sid == NS - 1)
        def _():
            pltpu.sync_copy(accum.at[pl.ds(NS * ROWS_PER_SUB, ROWS_TAIL)],
                            part_hbm.at[cid, pl.ds(NS * ROWS_PER_SUB, ROWS_TAIL)])

    fn = pl.kernel(
        body,
        out_type=jax.ShapeDtypeStruct((NC, N_NODES, D), jnp.float32),
        mesh=mesh,
        compiler_params=pltpu.CompilerParams(needs_layout_passes=False,
                                             use_tc_tiling_on_sc=False),
        scratch_types=[
            pltpu.VMEM((NCHUNK, CHUNK), jnp.int32),
            pltpu.VMEM((NCHUNK, CHUNK), jnp.int32),
            pltpu.VMEM((NBUF * WSTRIDE,), jnp.float32),
            pltpu.VMEM((NBUF, CHUNK, D), jnp.float32),
            pltpu.MemorySpace.VMEM_SHARED((N_NODES, D), jnp.float32),
        ] + [pltpu.SemaphoreType.DMA] * (2 * NBUF),
    )
    return fn(x, src4, dst4, w4, zeros)


def _mm_body(p_ref, w_ref, o_ref):
    acc = p_ref[0] + p_ref[1]
    o_ref[...] = jnp.maximum(
        jnp.dot(acc, w_ref[...], preferred_element_type=jnp.float32), 0.0)


def _tc_matmul_relu(partials, W):
    blk = 1000
    grid = N_NODES // blk
    return pl.pallas_call(
        _mm_body,
        grid=(grid,),
        in_specs=[
            pl.BlockSpec((NC, blk, D), lambda i: (0, i, 0)),
            pl.BlockSpec((D, D), lambda i: (0, 0)),
        ],
        out_specs=pl.BlockSpec((blk, D), lambda i: (i, 0)),
        out_shape=jax.ShapeDtypeStruct((N_NODES, D), jnp.float32),
    )(partials, W)


def kernel(x, edge_index, edge_weight, W):
    src4 = edge_index[0].astype(jnp.int32).reshape(NC, NS, NCHUNK, CHUNK)
    dst4 = edge_index[1].astype(jnp.int32).reshape(NC, NS, NCHUNK, CHUNK)
    w4 = edge_weight.astype(jnp.float32).reshape(NC, NS, NCHUNK, CHUNK)
    zeros = jnp.zeros((N_NODES, D), jnp.float32)
    partials = _sc_aggregate(x, src4, dst4, w4, zeros)
    return _tc_matmul_relu(partials, W)


# speedup vs baseline: 1.2414x; 1.2414x over previous
import jax
import jax.numpy as jnp
from jax import lax
from jax.experimental import pallas as pl
from jax.experimental.pallas import tpu as pltpu
from jax.experimental.pallas import tpu_sc as plsc

N_NODES = 10000
N_EDGES = 320000
D = 128

NC = 2
NS = 16
EPW = N_EDGES // (NC * NS)
CHUNK = 40
NCHUNK = EPW // CHUNK
NBUF = 5
PD = 3
WSTRIDE = 128
NMAIN = (NCHUNK // NBUF) * NBUF
ROWS_PER_SUB = 624
ROWS_TAIL = N_NODES - NS * ROWS_PER_SUB


def _sc_aggregate(x, src4, dst4, w4, zeros):
    mesh = plsc.VectorSubcoreMesh(core_axis_name="c", subcore_axis_name="s")

    def body(x_hbm, src_hbm, dst_hbm, w_hbm, z_hbm, part_hbm,
             src_v, dst_v, wflat, rows, accum,
             g0, g1, g2, g3, g4, s0, s1, s2, s3, s4):
        G = [g0, g1, g2, g3, g4]
        S = [s0, s1, s2, s3, s4]
        cid = lax.axis_index("c")
        sid = lax.axis_index("s")

        pltpu.sync_copy(z_hbm.at[pl.ds(sid * ROWS_PER_SUB, ROWS_PER_SUB)],
                        accum.at[pl.ds(sid * ROWS_PER_SUB, ROWS_PER_SUB)])

        @pl.when(sid == NS - 1)
        def _():
            pltpu.sync_copy(z_hbm.at[pl.ds(NS * ROWS_PER_SUB, ROWS_TAIL)],
                            accum.at[pl.ds(NS * ROWS_PER_SUB, ROWS_TAIL)])

        plsc.subcore_barrier()

        pltpu.sync_copy(src_hbm.at[cid, sid], src_v)
        pltpu.sync_copy(dst_hbm.at[cid, sid], dst_v)

        def gather_start(c, b):
            pltpu.async_copy(x_hbm.at[src_v.at[c]], rows.at[b], G[b])
            pltpu.async_copy(w_hbm.at[cid, sid, c],
                             wflat.at[pl.ds(b * WSTRIDE, CHUNK)], G[b])

        def gather_wait(b):
            pltpu.make_async_copy(
                x_hbm.at[pl.ds(0, CHUNK)], rows.at[b], G[b]).wait()
            pltpu.make_async_copy(
                w_hbm.at[0, 0, 0], wflat.at[pl.ds(b * WSTRIDE, CHUNK)],
                G[b]).wait()

        def scatter_start(c, b):
            pltpu.async_copy(rows.at[b], accum.at[dst_v.at[c]], S[b], add=True)

        def scatter_wait(b):
            pltpu.make_async_copy(
                rows.at[b], accum.at[pl.ds(0, CHUNK)], S[b]).wait()

        def step(cc, b, prefetch):
            gather_wait(b)
            if prefetch:
                b2 = (b + PD) % NBUF

                @pl.when(cc + PD < NCHUNK)
                def _():
                    @pl.when(cc + PD >= NBUF)
                    def _():
                        scatter_wait(b2)
                    gather_start(cc + PD, b2)

            rb = rows.at[b]

            @pl.loop(0, CHUNK, unroll=5)
            def edge_loop(e):
                ie = jnp.full((16,), b * WSTRIDE + e, jnp.int32)
                wvec = plsc.load_gather(wflat, [ie])
                for j in range(D // 16):
                    seg = rb[e, pl.ds(j * 16, 16)]
                    rb[e, pl.ds(j * 16, 16)] = seg * wvec

            scatter_start(cc, b)

        for p in range(PD):
            gather_start(p, p)

        @pl.loop(0, NMAIN, step=NBUF)
        def chunk_loop(c):
            for b in range(NBUF):
                step(c + b, b, True)

        for cc in range(NMAIN, NCHUNK):
            step(cc, cc % NBUF, True)

        for b in range(NBUF):
            scatter_wait(b)

        plsc.subcore_barrier()
        pltpu.sync_copy(accum.at[pl.ds(sid * ROWS_PER_SUB, ROWS_PER_SUB)],
                        part_hbm.at[cid, pl.ds(sid * ROWS_PER_SUB, ROWS_PER_SUB)])

        @pl.when(sid == NS - 1)
        def _():
            pltpu.sync_copy(accum.at[pl.ds(NS * ROWS_PER_SUB, ROWS_TAIL)],
                            part_hbm.at[cid, pl.ds(NS * ROWS_PER_SUB, ROWS_TAIL)])

    fn = pl.kernel(
        body,
        out_type=jax.ShapeDtypeStruct((NC, N_NODES, D), jnp.float32),
        mesh=mesh,
        compiler_params=pltpu.CompilerParams(needs_layout_passes=False,
                                             use_tc_tiling_on_sc=False),
        scratch_types=[
            pltpu.VMEM((NCHUNK, CHUNK), jnp.int32),
            pltpu.VMEM((NCHUNK, CHUNK), jnp.int32),
            pltpu.VMEM((NBUF * WSTRIDE,), jnp.float32),
            pltpu.VMEM((NBUF, CHUNK, D), jnp.float32),
            pltpu.MemorySpace.VMEM_SHARED((N_NODES, D), jnp.float32),
        ] + [pltpu.SemaphoreType.DMA] * (2 * NBUF),
    )
    return fn(x, src4, dst4, w4, zeros)


def _mm_body(p_ref, w_ref, o_ref):
    acc = p_ref[0] + p_ref[1]
    o_ref[...] = jnp.maximum(
        jnp.dot(acc, w_ref[...], preferred_element_type=jnp.float32), 0.0)


def _tc_matmul_relu(partials, W):
    blk = 1000
    grid = N_NODES // blk
    return pl.pallas_call(
        _mm_body,
        grid=(grid,),
        in_specs=[
            pl.BlockSpec((NC, blk, D), lambda i: (0, i, 0)),
            pl.BlockSpec((D, D), lambda i: (0, 0)),
        ],
        out_specs=pl.BlockSpec((blk, D), lambda i: (i, 0)),
        out_shape=jax.ShapeDtypeStruct((N_NODES, D), jnp.float32),
    )(partials, W)


def kernel(x, edge_index, edge_weight, W):
    src4 = edge_index[0].astype(jnp.int32).reshape(NC, NS, NCHUNK, CHUNK)
    dst4 = edge_index[1].astype(jnp.int32).reshape(NC, NS, NCHUNK, CHUNK)
    w4 = edge_weight.astype(jnp.float32).reshape(NC, NS, NCHUNK, CHUNK)
    zeros = jnp.zeros((N_NODES, D), jnp.float32)
    partials = _sc_aggregate(x, src4, dst4, w4, zeros)
    return _tc_matmul_relu(partials, W)
